# Initial kernel scaffold; baseline (speedup 1.0000x reference)
import jax
import jax.numpy as jnp
from jax.experimental import pallas as pl


def kernel(means3D, means2D, sh, opacities, scales, rotations, target_image, bg, viewmatrix, projmatrix, campos):
    raise NotImplementedError("write your pallas kernel here")


# speedup vs baseline: 1.5092x; 1.5092x over previous
import jax
import jax.numpy as jnp
from jax import lax
from jax.experimental import pallas as pl
from jax.experimental.pallas import tpu as pltpu

_TANX = 0.5
_TANY = 0.5
_C0 = 0.28209479177387814
_C1 = 0.4886025119029199
_C2 = [1.0925484305920792, -1.0925484305920792, 0.31539156525252005,
       -1.0925484305920792, 0.5462742152960396]
_C3 = [-0.5900435899266435, 2.890611442640554, -0.4570457994644658,
       0.3731763325901154, -0.4570457994644658, 1.445305721320277,
       -0.5900435899266435]

_HI = jax.lax.Precision.HIGHEST


def _bfr(u):
    ui = jax.lax.bitcast_convert_type(u, jnp.uint32)
    bias = jnp.uint32(0x7FFF) + ((ui >> 16) & jnp.uint32(1))
    return jax.lax.bitcast_convert_type((ui + bias) & jnp.uint32(0xFFFF0000),
                                        jnp.float32)


def _raster_body(m3r_ref, m3c_ref, shr_ref, opr_ref, scr_ref, rotr_ref,
                 tfl_ref, bgc_ref, vm_ref, pm_ref, cp_ref,
                 color_ref, alpha_ref, estc_ref, estw_ref, radii_ref,
                 *, Hpix, Wpix):
    P = m3c_ref.shape[1]
    N = tfl_ref.shape[2]
    f32 = jnp.float32

    m3 = m3r_ref[0]
    x = m3[0:1]; y = m3[1:2]; z = m3[2:3]
    vm = vm_ref[...]
    pm = pm_ref[...]

    def vs(mat, i, j):
        return mat[i:i + 1, j:j + 1]

    xb = _bfr(x); yb = _bfr(y); zb = _bfr(z)

    def dot4(mat, j):
        return ((xb * _bfr(vs(mat, 0, j)) + yb * _bfr(vs(mat, 1, j)))
                + zb * _bfr(vs(mat, 2, j))) + 1.0 * _bfr(vs(mat, 3, j))

    ph0 = dot4(pm, 0)
    ph1 = dot4(pm, 1)
    ph3 = dot4(pm, 3)
    pw = 1.0 / (ph3 + 1e-7)
    Wf = float(Wpix)
    Hf = float(Hpix)
    mx = ((ph0 * pw + 1.0) * Wf - 1.0) * 0.5
    my = ((ph1 * pw + 1.0) * Hf - 1.0) * 0.5

    tx = dot4(vm, 0)
    ty = dot4(vm, 1)
    tz = dot4(vm, 2)
    in_front = tz > 0.2
    tzs = jnp.where(in_front, tz, 1.0)
    fx = Wf / (2.0 * _TANX)
    fy = Hf / (2.0 * _TANY)
    limx = 1.3 * _TANX
    limy = 1.3 * _TANY
    txz = jnp.clip(tx / tzs, -limx, limx) * tzs
    tyz = jnp.clip(ty / tzs, -limy, limy) * tzs
    J00 = fx / tzs
    J02 = -fx * txz / (tzs * tzs)
    J11 = fy / tzs
    J12 = -fy * tyz / (tzs * tzs)

    q = rotr_ref[0]
    qw = q[0:1]; qx = q[1:2]; qy = q[2:3]; qz = q[3:4]
    qn = jnp.sqrt(qw * qw + qx * qx + qy * qy + qz * qz) + 1e-12
    qw = qw / qn; qx = qx / qn; qy = qy / qn; qz = qz / qn
    r00 = 1.0 - 2.0 * (qy * qy + qz * qz)
    r01 = 2.0 * (qx * qy - qw * qz)
    r02 = 2.0 * (qx * qz + qw * qy)
    r10 = 2.0 * (qx * qy + qw * qz)
    r11 = 1.0 - 2.0 * (qx * qx + qz * qz)
    r12 = 2.0 * (qy * qz - qw * qx)
    r20 = 2.0 * (qx * qz - qw * qy)
    r21 = 2.0 * (qy * qz + qw * qx)
    r22 = 1.0 - 2.0 * (qx * qx + qy * qy)
    sc = scr_ref[0]
    s0 = sc[0:1]; s1 = sc[1:2]; s2 = sc[2:3]
    mb = [[_bfr(r00 * s0), _bfr(r01 * s1), _bfr(r02 * s2)],
          [_bfr(r10 * s0), _bfr(r11 * s1), _bfr(r12 * s2)],
          [_bfr(r20 * s0), _bfr(r21 * s1), _bfr(r22 * s2)]]

    def d3(u, v):
        return (u[0] * v[0] + u[1] * v[1]) + u[2] * v[2]

    sg = [[d3(mb[i], mb[k]) for k in range(3)] for i in range(3)]
    sgb = [[_bfr(e) for e in row] for row in sg]

    zrow = jnp.zeros_like(J00)
    Jb = [[_bfr(J00), zrow, _bfr(J02)], [zrow, _bfr(J11), _bfr(J12)]]
    vmb = [[_bfr(vs(vm, i, j)) for j in range(3)] for i in range(3)]
    t = [[d3(Jb[r], vmb[j]) for j in range(3)] for r in range(2)]
    tb = [[_bfr(e) for e in row] for row in t]
    ts = [[d3(tb[r], [sgb[0][k], sgb[1][k], sgb[2][k]]) for k in range(3)]
          for r in range(2)]
    tsb = [[_bfr(e) for e in row] for row in ts]
    a_ = d3(tsb[0], tb[0]) + 0.3
    c_ = d3(tsb[1], tb[1]) + 0.3
    b_ = d3(tsb[0], tb[1])
    det = a_ * c_ - b_ * b_
    dets = jnp.where(jnp.abs(det) < 1e-10, 1e-10, det)
    cA = c_ / dets
    cB = -b_ / dets
    cC = a_ / dets
    mid = 0.5 * (a_ + c_)
    lam = mid + jnp.sqrt(jnp.maximum(0.1, mid * mid - det))
    valid_b = in_front & (det > 0)
    radii_f = jnp.where(valid_b, jnp.ceil(3.0 * jnp.sqrt(lam)), 0.0)
    valid = valid_b.astype(f32)

    cp = cp_ref[...]
    dx_ = x - cp[0:1, 0:1]
    dy_ = y - cp[0:1, 1:2]
    dz_ = z - cp[0:1, 2:3]
    dn = jnp.sqrt(dx_ * dx_ + dy_ * dy_ + dz_ * dz_) + 1e-12
    sx = dx_ / dn; sy = dy_ / dn; sz = dz_ / dn
    xx = sx * sx; yy = sy * sy; zz = sz * sz
    xy = sx * sy; yz = sy * sz; xz = sx * sz
    basis = [
        jnp.full_like(sx, _C0),
        -_C1 * sy, _C1 * sz, -_C1 * sx,
        _C2[0] * xy, _C2[1] * yz, _C2[2] * (2.0 * zz - xx - yy),
        _C2[3] * xz, _C2[4] * (xx - yy),
        _C3[0] * sy * (3.0 * xx - yy), _C3[1] * xy * sz,
        _C3[2] * sy * (4.0 * zz - xx - yy),
        _C3[3] * sz * (2.0 * zz - 3.0 * xx - 3.0 * yy),
        _C3[4] * sx * (4.0 * zz - xx - yy), _C3[5] * sz * (xx - yy),
        _C3[6] * sx * (xx - 3.0 * yy),
    ]
    shv = shr_ref[0]
    cols = []
    for ch in range(3):
        acc = basis[0] * shv[ch:ch + 1]
        for k in range(1, 16):
            acc = acc + basis[k] * shv[3 * k + ch:3 * k + ch + 1]
        cols.append(jnp.maximum(acc + 0.5, 0.0))

    op = opr_ref[0]

    m3c = m3c_ref[0]
    xc = _bfr(m3c[:, 0:1]); yc = _bfr(m3c[:, 1:2]); zc = _bfr(m3c[:, 2:3])
    tz_c = ((xc * _bfr(vs(vm, 0, 2)) + yc * _bfr(vs(vm, 1, 2)))
            + zc * _bfr(vs(vm, 2, 2))) + 1.0 * _bfr(vs(vm, 3, 2))
    iota_c = lax.broadcasted_iota(jnp.int32, (P, 1), 0).astype(f32)
    iota_r = lax.broadcasted_iota(jnp.int32, (1, P), 1).astype(f32)
    cmp = (tz_c < tz) | ((tz_c == tz) & (iota_c < iota_r))
    rank = jnp.sum(cmp.astype(f32), axis=0, keepdims=True)
    Gm = (rank == iota_c).astype(f32)

    attrs = jnp.concatenate(
        [mx, my, cA, cB, cC, op, valid, cols[0], cols[1], cols[2]], axis=0)
    sattr = lax.dot_general(Gm, attrs, (((1,), (1,)), ((), ())),
                            precision=_HI)

    ni = lax.broadcasted_iota(jnp.int32, (1, N), 1)
    pxr = (ni % Wpix).astype(f32)
    pyr = (ni // Wpix).astype(f32)
    tflv = tfl_ref[0]
    bgv = bgc_ref[0]

    K = 128
    Tc = jnp.ones((1, N), f32)
    outc = jnp.zeros((3, N), f32)
    outa = jnp.zeros((1, N), f32)
    est_list = []
    estw_list = []
    for ci in range(P // K):
        blk = sattr[ci * K:(ci + 1) * K]
        mxk = blk[:, 0:1]; myk = blk[:, 1:2]
        cAk = blk[:, 2:3]; cBk = blk[:, 3:4]; cCk = blk[:, 4:5]
        opk = blk[:, 5:6]; vak = blk[:, 6:7]
        colk = blk[:, 7:10]
        dxm = mxk - pxr
        dym = myk - pyr
        power = -0.5 * (cAk * dxm * dxm + cCk * dym * dym) - cBk * dxm * dym
        Ge = jnp.exp(jnp.minimum(power, 0.0))
        al = jnp.minimum(0.99, opk * Ge) * vak
        al = jnp.where(al < 1.0 / 255.0, 0.0, al)
        beta = 1.0 - al
        pprod = beta
        s = 1
        while s < K:
            pprod = pprod * jnp.concatenate(
                [jnp.ones((s, N), f32), pprod[:K - s]], axis=0)
            s *= 2
        excl = jnp.concatenate([jnp.ones((1, N), f32), pprod[:K - 1]], axis=0)
        w = al * (Tc * excl)
        outc = outc + lax.dot_general(colk, w, (((0,), (0,)), ((), ())),
                                      precision=_HI)
        outa = outa + jnp.sum(w, axis=0, keepdims=True)
        est_list.append(lax.dot_general(w, tflv, (((1,), (1,)), ((), ())),
                                        precision=_HI))
        estw_list.append(jnp.sum(w, axis=1, keepdims=True))
        Tc = Tc * pprod[K - 1:K]

    outc = outc + bgv * Tc
    est_s = jnp.concatenate(est_list, axis=0)
    estw_s = jnp.concatenate(estw_list, axis=0)
    est_o = lax.dot_general(Gm, est_s, (((0,), (0,)), ((), ())),
                            precision=_HI)
    estw_o = lax.dot_general(Gm, estw_s, (((0,), (0,)), ((), ())),
                             precision=_HI)

    color_ref[0] = outc
    alpha_ref[0] = outa
    estc_ref[0] = est_o
    estw_ref[0] = estw_o
    radii_ref[0] = radii_f.astype(jnp.int32)


def kernel(means3D, means2D, sh, opacities, scales, rotations, target_image,
           bg, viewmatrix, projmatrix, campos):
    f32 = jnp.float32
    B, P, _ = means3D.shape
    H, W = target_image.shape[2], target_image.shape[3]
    N = H * W
    m3c = means3D.astype(f32)
    m3r = m3c.transpose(0, 2, 1)
    shr = sh.astype(f32).reshape(B, P, 48).transpose(0, 2, 1)
    opr = opacities.astype(f32).transpose(0, 2, 1)
    scr = scales.astype(f32).transpose(0, 2, 1)
    rotr = rotations.astype(f32).transpose(0, 2, 1)
    tfl = target_image.astype(f32).reshape(B, 3, N)
    bgc = bg.astype(f32).reshape(B, 3, 1)
    vm = viewmatrix.astype(f32)
    pm = projmatrix.astype(f32)
    cp = campos.astype(f32).reshape(1, 3)

    def bspec(shape):
        return pl.BlockSpec(shape, lambda b: (b, 0, 0))

    def shared(shape):
        nd = len(shape)
        return pl.BlockSpec(shape, lambda b, _n=nd: (0,) * _n)

    import functools
    colorp, alphap, estc, estw, radii3 = pl.pallas_call(
        functools.partial(_raster_body, Hpix=H, Wpix=W),
        grid=(B,),
        in_specs=[
            bspec((1, 3, P)), bspec((1, P, 3)), bspec((1, 48, P)),
            bspec((1, 1, P)), bspec((1, 3, P)), bspec((1, 4, P)),
            bspec((1, 3, N)), bspec((1, 3, 1)),
            shared((4, 4)), shared((4, 4)), shared((1, 3)),
        ],
        out_specs=[
            bspec((1, 3, N)), bspec((1, 1, N)), bspec((1, P, 3)),
            bspec((1, P, 1)), bspec((1, 1, P)),
        ],
        out_shape=[
            jax.ShapeDtypeStruct((B, 3, N), f32),
            jax.ShapeDtypeStruct((B, 1, N), f32),
            jax.ShapeDtypeStruct((B, P, 3), f32),
            jax.ShapeDtypeStruct((B, P, 1), f32),
            jax.ShapeDtypeStruct((B, 1, P), jnp.int32),
        ],
    )(m3r, m3c, shr, opr, scr, rotr, tfl, bgc, vm, pm, cp)

    color = colorp.reshape(B, 3, H, W)
    alpha_img = alphap.reshape(B, 1, H, W)
    est_color = estc
    est_weight = estw.reshape(B, P)
    radii = radii3.reshape(B, P)
    return (color, alpha_img, est_color, est_weight, radii)


# speedup vs baseline: 1.8925x; 1.2540x over previous
import jax
import jax.numpy as jnp
from jax import lax
from jax.experimental import pallas as pl
from jax.experimental.pallas import tpu as pltpu

_TANX = 0.5
_TANY = 0.5
_C0 = 0.28209479177387814
_C1 = 0.4886025119029199
_C2 = [1.0925484305920792, -1.0925484305920792, 0.31539156525252005,
       -1.0925484305920792, 0.5462742152960396]
_C3 = [-0.5900435899266435, 2.890611442640554, -0.4570457994644658,
       0.3731763325901154, -0.4570457994644658, 1.445305721320277,
       -0.5900435899266435]

_HI = jax.lax.Precision.HIGHEST


def _bfr(u):
    ui = jax.lax.bitcast_convert_type(u, jnp.uint32)
    bias = jnp.uint32(0x7FFF) + ((ui >> 16) & jnp.uint32(1))
    return jax.lax.bitcast_convert_type((ui + bias) & jnp.uint32(0xFFFF0000),
                                        jnp.float32)


def _raster_body(m3r_ref, m3c_ref, shr_ref, opr_ref, scr_ref, rotr_ref,
                 tfl_ref, bgc_ref, vm_ref, pm_ref, cp_ref,
                 color_ref, alpha_ref, estc_ref, estw_ref, radii_ref,
                 *, Hpix, Wpix):
    P = m3c_ref.shape[1]
    N = tfl_ref.shape[2]
    f32 = jnp.float32

    m3 = m3r_ref[0]
    x = m3[0:1]; y = m3[1:2]; z = m3[2:3]
    vm = vm_ref[...]
    pm = pm_ref[...]

    def vs(mat, i, j):
        return mat[i:i + 1, j:j + 1]

    xb = _bfr(x); yb = _bfr(y); zb = _bfr(z)

    def dot4(mat, j):
        return ((xb * _bfr(vs(mat, 0, j)) + yb * _bfr(vs(mat, 1, j)))
                + zb * _bfr(vs(mat, 2, j))) + 1.0 * _bfr(vs(mat, 3, j))

    ph0 = dot4(pm, 0)
    ph1 = dot4(pm, 1)
    ph3 = dot4(pm, 3)
    pw = 1.0 / (ph3 + 1e-7)
    Wf = float(Wpix)
    Hf = float(Hpix)
    mx = ((ph0 * pw + 1.0) * Wf - 1.0) * 0.5
    my = ((ph1 * pw + 1.0) * Hf - 1.0) * 0.5

    tx = dot4(vm, 0)
    ty = dot4(vm, 1)
    tz = dot4(vm, 2)
    in_front = tz > 0.2
    tzs = jnp.where(in_front, tz, 1.0)
    fx = Wf / (2.0 * _TANX)
    fy = Hf / (2.0 * _TANY)
    limx = 1.3 * _TANX
    limy = 1.3 * _TANY
    txz = jnp.clip(tx / tzs, -limx, limx) * tzs
    tyz = jnp.clip(ty / tzs, -limy, limy) * tzs
    J00 = fx / tzs
    J02 = -fx * txz / (tzs * tzs)
    J11 = fy / tzs
    J12 = -fy * tyz / (tzs * tzs)

    q = rotr_ref[0]
    qw = q[0:1]; qx = q[1:2]; qy = q[2:3]; qz = q[3:4]
    qn = jnp.sqrt(qw * qw + qx * qx + qy * qy + qz * qz) + 1e-12
    qw = qw / qn; qx = qx / qn; qy = qy / qn; qz = qz / qn
    r00 = 1.0 - 2.0 * (qy * qy + qz * qz)
    r01 = 2.0 * (qx * qy - qw * qz)
    r02 = 2.0 * (qx * qz + qw * qy)
    r10 = 2.0 * (qx * qy + qw * qz)
    r11 = 1.0 - 2.0 * (qx * qx + qz * qz)
    r12 = 2.0 * (qy * qz - qw * qx)
    r20 = 2.0 * (qx * qz - qw * qy)
    r21 = 2.0 * (qy * qz + qw * qx)
    r22 = 1.0 - 2.0 * (qx * qx + qy * qy)
    sc = scr_ref[0]
    s0 = sc[0:1]; s1 = sc[1:2]; s2 = sc[2:3]
    mb = [[_bfr(r00 * s0), _bfr(r01 * s1), _bfr(r02 * s2)],
          [_bfr(r10 * s0), _bfr(r11 * s1), _bfr(r12 * s2)],
          [_bfr(r20 * s0), _bfr(r21 * s1), _bfr(r22 * s2)]]

    def d3(u, v):
        return (u[0] * v[0] + u[1] * v[1]) + u[2] * v[2]

    sg = [[d3(mb[i], mb[k]) for k in range(3)] for i in range(3)]
    sgb = [[_bfr(e) for e in row] for row in sg]

    zrow = jnp.zeros_like(J00)
    Jb = [[_bfr(J00), zrow, _bfr(J02)], [zrow, _bfr(J11), _bfr(J12)]]
    vmb = [[_bfr(vs(vm, i, j)) for j in range(3)] for i in range(3)]
    t = [[d3(Jb[r], vmb[j]) for j in range(3)] for r in range(2)]
    tb = [[_bfr(e) for e in row] for row in t]
    ts = [[d3(tb[r], [sgb[0][k], sgb[1][k], sgb[2][k]]) for k in range(3)]
          for r in range(2)]
    tsb = [[_bfr(e) for e in row] for row in ts]
    a_ = d3(tsb[0], tb[0]) + 0.3
    c_ = d3(tsb[1], tb[1]) + 0.3
    b_ = d3(tsb[0], tb[1])
    det = a_ * c_ - b_ * b_
    dets = jnp.where(jnp.abs(det) < 1e-10, 1e-10, det)
    cA = c_ / dets
    cB = -b_ / dets
    cC = a_ / dets
    mid = 0.5 * (a_ + c_)
    lam = mid + jnp.sqrt(jnp.maximum(0.1, mid * mid - det))
    valid_b = in_front & (det > 0)
    radii_f = jnp.where(valid_b, jnp.ceil(3.0 * jnp.sqrt(lam)), 0.0)
    valid = valid_b.astype(f32)

    cp = cp_ref[...]
    dx_ = x - cp[0:1, 0:1]
    dy_ = y - cp[0:1, 1:2]
    dz_ = z - cp[0:1, 2:3]
    dn = jnp.sqrt(dx_ * dx_ + dy_ * dy_ + dz_ * dz_) + 1e-12
    sx = dx_ / dn; sy = dy_ / dn; sz = dz_ / dn
    xx = sx * sx; yy = sy * sy; zz = sz * sz
    xy = sx * sy; yz = sy * sz; xz = sx * sz
    basis = [
        jnp.full_like(sx, _C0),
        -_C1 * sy, _C1 * sz, -_C1 * sx,
        _C2[0] * xy, _C2[1] * yz, _C2[2] * (2.0 * zz - xx - yy),
        _C2[3] * xz, _C2[4] * (xx - yy),
        _C3[0] * sy * (3.0 * xx - yy), _C3[1] * xy * sz,
        _C3[2] * sy * (4.0 * zz - xx - yy),
        _C3[3] * sz * (2.0 * zz - 3.0 * xx - 3.0 * yy),
        _C3[4] * sx * (4.0 * zz - xx - yy), _C3[5] * sz * (xx - yy),
        _C3[6] * sx * (xx - 3.0 * yy),
    ]
    shv = shr_ref[0]
    cols = []
    for ch in range(3):
        acc = basis[0] * shv[ch:ch + 1]
        for k in range(1, 16):
            acc = acc + basis[k] * shv[3 * k + ch:3 * k + ch + 1]
        cols.append(jnp.maximum(acc + 0.5, 0.0))

    op = opr_ref[0] * valid

    m3c = m3c_ref[0]
    xc = _bfr(m3c[:, 0:1]); yc = _bfr(m3c[:, 1:2]); zc = _bfr(m3c[:, 2:3])
    tz_c = ((xc * _bfr(vs(vm, 0, 2)) + yc * _bfr(vs(vm, 1, 2)))
            + zc * _bfr(vs(vm, 2, 2))) + 1.0 * _bfr(vs(vm, 3, 2))
    iota_c = lax.broadcasted_iota(jnp.int32, (P, 1), 0).astype(f32)
    iota_r = lax.broadcasted_iota(jnp.int32, (1, P), 1).astype(f32)
    cmp = (tz_c < tz) | ((tz_c == tz) & (iota_c < iota_r))
    rank = jnp.sum(cmp.astype(f32), axis=0, keepdims=True)
    Gm = (rank == iota_c).astype(f32)

    attrs = jnp.concatenate(
        [mx, my, cA, cB, cC, op, cols[0], cols[1], cols[2]], axis=0)
    sattr = lax.dot_general(Gm, attrs, (((1,), (1,)), ((), ())),
                            precision=_HI)

    ni = lax.broadcasted_iota(jnp.int32, (1, N), 1)
    pxr = (ni % Wpix).astype(f32)
    pyr = (ni // Wpix).astype(f32)
    tflv = tfl_ref[0]
    bgv = bgc_ref[0]

    K = 128
    Tc = jnp.ones((1, N), f32)
    outc = jnp.zeros((3, N), f32)
    outa = jnp.zeros((1, N), f32)
    est_list = []
    estw_list = []
    for ci in range(P // K):
        blk = sattr[ci * K:(ci + 1) * K]
        mxk = blk[:, 0:1]; myk = blk[:, 1:2]
        cAk = blk[:, 2:3]; cBk = blk[:, 3:4]; cCk = blk[:, 4:5]
        opk = blk[:, 5:6]
        colk = blk[:, 6:9]
        dxm = mxk - pxr
        dym = myk - pyr
        power = -0.5 * (cAk * dxm * dxm + cCk * dym * dym) - cBk * dxm * dym
        Ge = jnp.exp(jnp.minimum(power, 0.0))
        al = jnp.minimum(0.99, opk * Ge)
        al = jnp.where(al < 1.0 / 255.0, 0.0, al)
        beta = 1.0 - al
        pprod = beta
        s = 1
        while s < K:
            pprod = pprod * jnp.concatenate(
                [jnp.ones((s, N), f32), pprod[:K - s]], axis=0)
            s *= 2
        excl = jnp.concatenate([jnp.ones((1, N), f32), pprod[:K - 1]], axis=0)
        w = al * (Tc * excl)
        outc = outc + lax.dot_general(colk, w, (((0,), (0,)), ((), ())))
        outa = outa + jnp.sum(w, axis=0, keepdims=True)
        est_list.append(lax.dot_general(w, tflv, (((1,), (1,)), ((), ()))))
        estw_list.append(jnp.sum(w, axis=1, keepdims=True))
        Tc = Tc * pprod[K - 1:K]

    outc = outc + bgv * Tc
    est_s = jnp.concatenate(est_list, axis=0)
    estw_s = jnp.concatenate(estw_list, axis=0)
    est_o = lax.dot_general(Gm, est_s, (((0,), (0,)), ((), ())),
                            precision=_HI)
    estw_o = lax.dot_general(Gm, estw_s, (((0,), (0,)), ((), ())),
                             precision=_HI)

    color_ref[0] = outc
    alpha_ref[0] = outa
    estc_ref[0] = est_o
    estw_ref[0] = estw_o
    radii_ref[0] = radii_f.astype(jnp.int32)


def kernel(means3D, means2D, sh, opacities, scales, rotations, target_image,
           bg, viewmatrix, projmatrix, campos):
    f32 = jnp.float32
    B, P, _ = means3D.shape
    H, W = target_image.shape[2], target_image.shape[3]
    N = H * W
    m3c = means3D.astype(f32)
    m3r = m3c.transpose(0, 2, 1)
    shr = sh.astype(f32).reshape(B, P, 48).transpose(0, 2, 1)
    opr = opacities.astype(f32).transpose(0, 2, 1)
    scr = scales.astype(f32).transpose(0, 2, 1)
    rotr = rotations.astype(f32).transpose(0, 2, 1)
    tfl = target_image.astype(f32).reshape(B, 3, N)
    bgc = bg.astype(f32).reshape(B, 3, 1)
    vm = viewmatrix.astype(f32)
    pm = projmatrix.astype(f32)
    cp = campos.astype(f32).reshape(1, 3)

    def bspec(shape):
        return pl.BlockSpec(shape, lambda b: (b, 0, 0))

    def shared(shape):
        nd = len(shape)
        return pl.BlockSpec(shape, lambda b, _n=nd: (0,) * _n)

    import functools
    colorp, alphap, estc, estw, radii3 = pl.pallas_call(
        functools.partial(_raster_body, Hpix=H, Wpix=W),
        grid=(B,),
        in_specs=[
            bspec((1, 3, P)), bspec((1, P, 3)), bspec((1, 48, P)),
            bspec((1, 1, P)), bspec((1, 3, P)), bspec((1, 4, P)),
            bspec((1, 3, N)), bspec((1, 3, 1)),
            shared((4, 4)), shared((4, 4)), shared((1, 3)),
        ],
        out_specs=[
            bspec((1, 3, N)), bspec((1, 1, N)), bspec((1, P, 3)),
            bspec((1, P, 1)), bspec((1, 1, P)),
        ],
        out_shape=[
            jax.ShapeDtypeStruct((B, 3, N), f32),
            jax.ShapeDtypeStruct((B, 1, N), f32),
            jax.ShapeDtypeStruct((B, P, 3), f32),
            jax.ShapeDtypeStruct((B, P, 1), f32),
            jax.ShapeDtypeStruct((B, 1, P), jnp.int32),
        ],
    )(m3r, m3c, shr, opr, scr, rotr, tfl, bgc, vm, pm, cp)

    color = colorp.reshape(B, 3, H, W)
    alpha_img = alphap.reshape(B, 1, H, W)
    est_color = estc
    est_weight = estw.reshape(B, P)
    radii = radii3.reshape(B, P)
    return (color, alpha_img, est_color, est_weight, radii)
